# baseline (speedup 1.0000x reference)
import jax
import jax.numpy as jnp
from jax import lax
from jax.experimental import pallas as pl
from jax.experimental.pallas import tpu as pltpu

_N = 10000
_ROWW = 85
_GRID = 10
_BLK = _N // _GRID
_FOLD = _BLK // 8


def _body(x_ref, hbm_ref, o_ref, vaccs_ref, conf_ref, xs_ref, sem):
    i = pl.program_id(0)

    x = x_ref[0, :, :]
    s = x.reshape(_FOLD, 8, _ROWW)
    addf = jnp.sum(s, axis=0)
    minf = jnp.min(s, axis=0)
    lane8 = lax.broadcasted_iota(jnp.int32, (8, _ROWW), 1)
    confmin = jnp.min(jnp.where(lane8 == 4, minf, jnp.inf))
    vaccs_ref[i] = addf
    conf_ref[0, i] = confmin

    @pl.when(i == _GRID - 1)
    def _():
        fb = _GRID
        for j in reversed(range(_GRID)):
            fb = jnp.where(conf_ref[0, j] < 0.0, j, fb)

        blki = lax.broadcasted_iota(jnp.int32, (_GRID, 8, _ROWW), 0)
        lane3 = lax.broadcasted_iota(jnp.int32, (_GRID, 8, _ROWW), 2)
        keep = jnp.logical_and(blki < fb, lane3 < 5)
        tot = jnp.sum(jnp.where(keep, vaccs_ref[...], 0.0))

        fbc = jnp.minimum(fb, _GRID - 1)
        cp = pltpu.make_async_copy(
            hbm_ref.at[0, pl.ds(fbc * _BLK, _BLK), :], xs_ref, sem)
        cp.start()
        cp.wait()
        xs = xs_ref[...]
        rowi = lax.broadcasted_iota(jnp.int32, (_BLK, _ROWW), 0)
        lane = lax.broadcasted_iota(jnp.int32, (_BLK, _ROWW), 1)
        conf_bad = jnp.logical_and(lane == 4, xs < 0.0)
        r_bad = jnp.min(jnp.where(conf_bad, rowi, _BLK))
        m = jnp.logical_and(lane < 5, rowi < r_bad)
        part = jnp.sum(jnp.where(m, xs, 0.0))
        tot = tot + jnp.where(fb < _GRID, part, 0.0)

        o_ref[...] = jnp.full((1, 1), tot, jnp.float32)


def kernel(data):
    out = pl.pallas_call(
        _body,
        grid=(_GRID,),
        in_specs=[pl.BlockSpec((1, _BLK, _ROWW), lambda i: (0, i, 0)),
                  pl.BlockSpec(memory_space=pltpu.MemorySpace.HBM)],
        out_specs=pl.BlockSpec((1, 1), lambda i: (0, 0)),
        out_shape=jax.ShapeDtypeStruct((1, 1), jnp.float32),
        scratch_shapes=[pltpu.VMEM((_GRID, 8, _ROWW), jnp.float32),
                        pltpu.SMEM((1, _GRID), jnp.float32),
                        pltpu.VMEM((_BLK, _ROWW), jnp.float32),
                        pltpu.SemaphoreType.DMA],
    )(data, data)
    return out[0, 0]


# speedup vs baseline: 6.4880x; 6.4880x over previous
import jax
import jax.numpy as jnp
from jax import lax
from jax.experimental import pallas as pl
from jax.experimental.pallas import tpu as pltpu

_N = 10000
_NPAD = 10112


def _body(x0_ref, x1_ref, x2_ref, x3_ref, x4_ref, o_ref):
    x4 = x4_ref[...]
    c = lax.broadcasted_iota(jnp.int32, (_NPAD,), 0)
    badm = jnp.logical_and(c < _N, x4 < 0.0)
    cbad = jnp.min(jnp.where(badm, c, _N))
    keep = c < cbad
    s = x0_ref[...] + x1_ref[...] + x2_ref[...] + x3_ref[...] + x4
    tot = jnp.sum(jnp.where(keep, s, 0.0))
    o_ref[...] = jnp.full((1, 1), tot, jnp.float32)


def kernel(data):
    cols = [data[0, :_NPAD, i] for i in range(5)]
    out = pl.pallas_call(
        _body,
        grid=(1,),
        in_specs=[pl.BlockSpec((_NPAD,), lambda i: (0,))] * 5,
        out_specs=pl.BlockSpec((1, 1), lambda i: (0, 0)),
        out_shape=jax.ShapeDtypeStruct((1, 1), jnp.float32),
    )(*cols)
    return out[0, 0]


# speedup vs baseline: 6.6721x; 1.0284x over previous
import jax
import jax.numpy as jnp
from jax import lax
from jax.experimental import pallas as pl
from jax.experimental.pallas import tpu as pltpu

_N = 10000
_NPAD = 10112


def _body(x0_ref, x1_ref, x2_ref, x3_ref, x4_ref, o_ref):
    pos = (lax.broadcasted_iota(jnp.int32, (79, 128), 0) * 128
           + lax.broadcasted_iota(jnp.int32, (79, 128), 1))
    x4 = x4_ref[...]
    badm = jnp.logical_and(pos < _N, x4 < 0.0)
    cbad = jnp.min(jnp.where(badm, pos, _N))
    s = x0_ref[...] + x1_ref[...] + x2_ref[...] + x3_ref[...] + x4
    tot = jnp.sum(jnp.where(pos < cbad, s, 0.0))
    o_ref[...] = jnp.full((1, 1), tot, jnp.float32)


def kernel(data):
    cols = [data[0, :_NPAD, i].reshape(79, 128) for i in range(5)]
    out = pl.pallas_call(
        _body,
        grid=(1,),
        in_specs=[pl.BlockSpec((79, 128), lambda i: (0, 0))] * 5,
        out_specs=pl.BlockSpec((1, 1), lambda i: (0, 0)),
        out_shape=jax.ShapeDtypeStruct((1, 1), jnp.float32),
    )(*cols)
    return out[0, 0]


# speedup vs baseline: 7.3303x; 1.0987x over previous
import jax
import jax.numpy as jnp
from jax import lax
from jax.experimental import pallas as pl
from jax.experimental.pallas import tpu as pltpu

_N = 10000
_NPAD = 10240


def _body(x_ref, o_ref):
    pos = (lax.broadcasted_iota(jnp.int32, (80, 128), 0) * 128
           + lax.broadcasted_iota(jnp.int32, (80, 128), 1))
    x4 = x_ref[4]
    badm = jnp.logical_and(pos < _N, x4 < 0.0)
    cbad = jnp.min(jnp.where(badm, pos, _N))
    s = x_ref[0] + x_ref[1] + x_ref[2] + x_ref[3] + x4
    tot = jnp.sum(jnp.where(pos < cbad, s, 0.0))
    o_ref[...] = jnp.full((1, 1), tot, jnp.float32)


def kernel(data):
    big = jnp.transpose(data[0, :_NPAD, 0:5])
    out = pl.pallas_call(
        _body,
        grid=(1,),
        in_specs=[pl.BlockSpec((5, 80, 128), lambda i: (0, 0, 0))],
        out_specs=pl.BlockSpec((1, 1), lambda i: (0, 0)),
        out_shape=jax.ShapeDtypeStruct((1, 1), jnp.float32),
    )(big.reshape(5, 80, 128))
    return out[0, 0]


# speedup vs baseline: 7.3756x; 1.0062x over previous
import jax
import jax.numpy as jnp
from jax import lax
from jax.experimental import pallas as pl
from jax.experimental.pallas import tpu as pltpu

_N = 10000
_NPAD = 10240


def _body(x_ref, o_ref):
    pos = (lax.broadcasted_iota(jnp.int32, (80, 128), 0) * 128
           + lax.broadcasted_iota(jnp.int32, (80, 128), 1))
    x4 = x_ref[4]
    badm = jnp.logical_and(pos < _N, x4 < 0.0)
    cbad = jnp.min(jnp.where(badm, pos, _N))
    s = x_ref[0] + x_ref[1] + x_ref[2] + x_ref[3] + x4
    tot = jnp.sum(jnp.where(pos < cbad, s, 0.0))
    o_ref[...] = jnp.full((1, 1), tot, jnp.float32)


def kernel(data):
    big = jnp.transpose(data[0, :_NPAD, 0:5])
    out = pl.pallas_call(
        _body,
        grid=(1,),
        in_specs=[pl.BlockSpec(memory_space=pltpu.MemorySpace.VMEM)],
        out_specs=pl.BlockSpec((1, 1), lambda i: (0, 0)),
        out_shape=jax.ShapeDtypeStruct((1, 1), jnp.float32),
    )(big.reshape(5, 80, 128))
    return out[0, 0]
